# baseline (speedup 1.0000x reference)
import jax
import jax.numpy as jnp
from jax.experimental import pallas as pl
from jax.experimental.pallas import tpu as pltpu


def _fused_kernel(x_ref, w_ref, b_ref, o_ref):
    z = jax.lax.dot_general(
        x_ref[:], w_ref[:], (((1,), (1,)), ((), ())),
        preferred_element_type=jnp.float32)
    z = jax.nn.sigmoid(z + b_ref[:])
    m = jnp.max(z, axis=1, keepdims=True)
    lse = m + jnp.log(jnp.sum(jnp.exp(z - m), axis=1, keepdims=True))
    o_ref[:] = z - lse


def kernel(x, edge_index, W, b):
    del edge_index
    N, D = x.shape
    C = W.shape[0]
    b2 = b.reshape(1, C)
    BR = 1000
    return pl.pallas_call(
        _fused_kernel,
        grid=(N // BR,),
        compiler_params=pltpu.CompilerParams(
            dimension_semantics=("parallel",)),
        in_specs=[
            pl.BlockSpec((BR, D), lambda i: (i, 0)),
            pl.BlockSpec((C, D), lambda i: (0, 0)),
            pl.BlockSpec((1, C), lambda i: (0, 0)),
        ],
        out_specs=pl.BlockSpec((BR, C), lambda i: (i, 0)),
        out_shape=jax.ShapeDtypeStruct((N, C), jnp.float32),
    )(x, W, b2)


# speedup vs baseline: 1.1413x; 1.1413x over previous
import jax
import jax.numpy as jnp
from jax.experimental import pallas as pl
from jax.experimental.pallas import tpu as pltpu

_K = 5
_BR = 2000
_SUB = _BR // _K


def _fused_kernel(*refs):
    xrefs = refs[:_K]
    w_ref, b_ref, o_ref = refs[_K:]
    for j in range(_K):
        z = jax.lax.dot_general(
            xrefs[j][:], w_ref[:], (((1,), (1,)), ((), ())),
            preferred_element_type=jnp.float32)
        z = jax.nn.sigmoid(z + b_ref[:])
        m = jnp.max(z, axis=1, keepdims=True)
        lse = m + jnp.log(jnp.sum(jnp.exp(z - m), axis=1, keepdims=True))
        o_ref[j * _SUB:(j + 1) * _SUB, :] = z - lse


def kernel(x, edge_index, W, b):
    del edge_index
    N, D = x.shape
    C = W.shape[0]
    b2 = b.reshape(1, C)
    in_specs = [
        pl.BlockSpec((_SUB, D), (lambda i, j=j: (i * _K + j, 0)))
        for j in range(_K)
    ]
    in_specs.append(pl.BlockSpec((C, D), lambda i: (0, 0)))
    in_specs.append(pl.BlockSpec((1, C), lambda i: (0, 0)))
    return pl.pallas_call(
        _fused_kernel,
        grid=(N // _BR,),
        in_specs=in_specs,
        out_specs=pl.BlockSpec((_BR, C), lambda i: (i, 0)),
        out_shape=jax.ShapeDtypeStruct((N, C), jnp.float32),
        compiler_params=pltpu.CompilerParams(
            dimension_semantics=("arbitrary",)),
    )(*([x] * _K), W, b2)
